# Initial kernel scaffold; baseline (speedup 1.0000x reference)
import jax
import jax.numpy as jnp
from jax.experimental import pallas as pl


def kernel(x, table):
    raise NotImplementedError("write your pallas kernel here")


# speedup vs baseline: 6.4932x; 6.4932x over previous
import functools

import jax
import jax.numpy as jnp
from jax import lax
from jax.experimental import pallas as pl
from jax.experimental.pallas import tpu as pltpu
from jax.experimental.pallas import tpu_sc as plsc

VOCAB = 1000
D_EMB = 128
BATCH = 4096
HIST = 200

NC, NS = 2, 16
NW = NC * NS
ROWS = BATCH * HIST
RPW = ROWS // NW
CHUNK = 128
NCHUNK = RPW // CHUNK
NBUF = 2


def _sc_body(table_hbm, idx_hbm, out_hbm, idx_v, buf0, buf1, g0, g1, o0, o1):
    wid = lax.axis_index("s") * NC + lax.axis_index("c")
    bufs = (buf0, buf1)
    gsems = (g0, g1)
    osems = (o0, o1)
    cbase = wid * NCHUNK

    pltpu.sync_copy(idx_hbm.at[wid], idx_v)

    def g_start(g, b):
        pltpu.async_copy(table_hbm.at[idx_v.at[g]], bufs[b], gsems[b])

    def g_wait(g, b):
        pltpu.make_async_copy(table_hbm.at[idx_v.at[g]], bufs[b], gsems[b]).wait()

    def o_start(g, b):
        pltpu.async_copy(bufs[b], out_hbm.at[cbase + g], osems[b])

    def o_wait(g, b):
        pltpu.make_async_copy(bufs[b], out_hbm.at[cbase + g], osems[b]).wait()

    for b in range(NBUF):
        g_start(b, b)

    def loop_body(i, carry):
        for b in range(NBUF):
            g = i * NBUF + b
            g_wait(g, b)
            o_start(g, b)

            @pl.when(g + NBUF < NCHUNK)
            def _():
                o_wait(g, b)
                g_start(g + NBUF, b)

        return carry

    lax.fori_loop(0, NCHUNK // NBUF, loop_body, None)

    for b in range(NBUF):
        o_wait(NCHUNK - NBUF + b, b)


_sc_gather = functools.partial(
    pl.kernel,
    out_type=jax.ShapeDtypeStruct((NW * NCHUNK, CHUNK, D_EMB), jnp.float32),
    mesh=plsc.VectorSubcoreMesh(core_axis_name="c", subcore_axis_name="s"),
    scratch_types=[
        pltpu.VMEM((NCHUNK, CHUNK), jnp.int32),
        pltpu.VMEM((CHUNK, D_EMB), jnp.float32),
        pltpu.VMEM((CHUNK, D_EMB), jnp.float32),
        pltpu.SemaphoreType.DMA,
        pltpu.SemaphoreType.DMA,
        pltpu.SemaphoreType.DMA,
        pltpu.SemaphoreType.DMA,
    ],
)(_sc_body)


def kernel(x, table):
    idx = x.astype(jnp.int32).reshape(NW, NCHUNK, CHUNK)
    out = _sc_gather(table, idx)
    return out.reshape(BATCH, HIST, D_EMB)


# speedup vs baseline: 6.5224x; 1.0045x over previous
import functools

import jax
import jax.numpy as jnp
from jax import lax
from jax.experimental import pallas as pl
from jax.experimental.pallas import tpu as pltpu
from jax.experimental.pallas import tpu_sc as plsc

VOCAB = 1000
D_EMB = 128
BATCH = 4096
HIST = 200

NC, NS = 2, 16
NW = NC * NS
ROWS = BATCH * HIST
RPW = ROWS // NW
CHUNK = 128
NCHUNK = RPW // CHUNK
NBUF = 4


def _sc_body(table_hbm, idx_hbm, out_hbm, idx_v,
             buf0, buf1, buf2, buf3, g0, g1, g2, g3, o0, o1, o2, o3):
    wid = lax.axis_index("s") * NC + lax.axis_index("c")
    bufs = (buf0, buf1, buf2, buf3)
    gsems = (g0, g1, g2, g3)
    osems = (o0, o1, o2, o3)
    cbase = wid * NCHUNK

    pltpu.sync_copy(idx_hbm.at[wid], idx_v)

    def g_start(g, b):
        pltpu.async_copy(table_hbm.at[idx_v.at[g]], bufs[b], gsems[b])

    def g_wait(g, b):
        pltpu.make_async_copy(table_hbm.at[idx_v.at[g]], bufs[b], gsems[b]).wait()

    def o_start(g, b):
        pltpu.async_copy(bufs[b], out_hbm.at[cbase + g], osems[b])

    def o_wait(g, b):
        pltpu.make_async_copy(bufs[b], out_hbm.at[cbase + g], osems[b]).wait()

    g_start(0, 0)
    g_start(1, 1)

    for g in range(NBUF):
        g_wait(g, g % NBUF)
        o_start(g, g % NBUF)
        if g >= 2:
            o_wait(g - 2, (g + 2) % NBUF)
        g_start(g + 2, (g + 2) % NBUF)

    def loop_body(i, carry):
        for b in range(NBUF):
            g = i * NBUF + b
            b2 = (b + 2) % NBUF
            g_wait(g, b)
            o_start(g, b)
            o_wait(g - 2, b2)
            g_start(g + 2, b2)
        return carry

    lax.fori_loop(1, NCHUNK // NBUF - 1, loop_body, None)

    for b in range(NBUF):
        g = NCHUNK - NBUF + b
        g_wait(g, b)
        o_start(g, b)
        if g + 2 < NCHUNK:
            o_wait(g - 2, (g + 2) % NBUF)
            g_start(g + 2, (g + 2) % NBUF)

    for b in range(NBUF):
        g = NCHUNK - NBUF + b
        o_wait(g, b)


_sc_gather = functools.partial(
    pl.kernel,
    out_type=jax.ShapeDtypeStruct((NW * NCHUNK, CHUNK, D_EMB), jnp.float32),
    mesh=plsc.VectorSubcoreMesh(core_axis_name="c", subcore_axis_name="s"),
    scratch_types=(
        [pltpu.VMEM((NCHUNK, CHUNK), jnp.int32)]
        + [pltpu.VMEM((CHUNK, D_EMB), jnp.float32)] * NBUF
        + [pltpu.SemaphoreType.DMA] * (2 * NBUF)
    ),
)(_sc_body)


def kernel(x, table):
    idx = x.astype(jnp.int32).reshape(NW, NCHUNK, CHUNK)
    out = _sc_gather(table, idx)
    return out.reshape(BATCH, HIST, D_EMB)


# speedup vs baseline: 15.7639x; 2.4169x over previous
import functools

import jax
import jax.numpy as jnp
from jax import lax
from jax.experimental import pallas as pl
from jax.experimental.pallas import tpu as pltpu
from jax.experimental.pallas import tpu_sc as plsc

VOCAB = 1000
D_EMB = 128
BATCH = 4096
HIST = 200

NC, NS = 2, 16
NW = NC * NS
ROWS = BATCH * HIST
RPW = ROWS // NW
CHUNK = 128
NCHUNK = RPW // CHUNK
NBUF = 4


def _sc_body(table_hbm, idx_hbm, out_hbm, table_sp, idx_v,
             buf0, buf1, buf2, buf3, g0, g1, g2, g3, o0, o1, o2, o3):
    sid = lax.axis_index("s")
    wid = sid * NC + lax.axis_index("c")
    bufs = (buf0, buf1, buf2, buf3)
    gsems = (g0, g1, g2, g3)
    osems = (o0, o1, o2, o3)
    cbase = wid * NCHUNK

    @pl.when(sid == 0)
    def _():
        pltpu.sync_copy(table_hbm, table_sp)

    pltpu.sync_copy(idx_hbm.at[wid], idx_v)
    plsc.subcore_barrier()

    def g_start(g, b):
        pltpu.async_copy(table_sp.at[idx_v.at[g]], bufs[b], gsems[b])

    def g_wait(g, b):
        pltpu.make_async_copy(table_sp.at[idx_v.at[g]], bufs[b], gsems[b]).wait()

    def o_start(g, b):
        pltpu.async_copy(bufs[b], out_hbm.at[cbase + g], osems[b])

    def o_wait(g, b):
        pltpu.make_async_copy(bufs[b], out_hbm.at[cbase + g], osems[b]).wait()

    g_start(0, 0)
    g_start(1, 1)

    for g in range(NBUF):
        g_wait(g, g % NBUF)
        o_start(g, g % NBUF)
        if g >= 2:
            o_wait(g - 2, (g + 2) % NBUF)
        g_start(g + 2, (g + 2) % NBUF)

    def loop_body(i, carry):
        for b in range(NBUF):
            g = i * NBUF + b
            b2 = (b + 2) % NBUF
            g_wait(g, b)
            o_start(g, b)
            o_wait(g - 2, b2)
            g_start(g + 2, b2)
        return carry

    lax.fori_loop(1, NCHUNK // NBUF - 1, loop_body, None)

    for b in range(NBUF):
        g = NCHUNK - NBUF + b
        g_wait(g, b)
        o_start(g, b)
        if g + 2 < NCHUNK:
            o_wait(g - 2, (g + 2) % NBUF)
            g_start(g + 2, (g + 2) % NBUF)

    for b in range(NBUF):
        g = NCHUNK - NBUF + b
        o_wait(g, b)


_sc_gather = functools.partial(
    pl.kernel,
    out_type=jax.ShapeDtypeStruct((NW * NCHUNK, CHUNK, D_EMB), jnp.float32),
    mesh=plsc.VectorSubcoreMesh(core_axis_name="c", subcore_axis_name="s"),
    scratch_types=(
        [pltpu.VMEM_SHARED((VOCAB, D_EMB), jnp.float32)]
        + [pltpu.VMEM((NCHUNK, CHUNK), jnp.int32)]
        + [pltpu.VMEM((CHUNK, D_EMB), jnp.float32)] * NBUF
        + [pltpu.SemaphoreType.DMA] * (2 * NBUF)
    ),
)(_sc_body)


def kernel(x, table):
    idx = x.astype(jnp.int32).reshape(NW, NCHUNK, CHUNK)
    out = _sc_gather(table, idx)
    return out.reshape(BATCH, HIST, D_EMB)
